# Initial kernel scaffold; baseline (speedup 1.0000x reference)
import jax
import jax.numpy as jnp
from jax.experimental import pallas as pl


def kernel(price_td, price_range_idx, description_idx, sku_idx, heir_idx, visual, price_range_table, desc_table, sku_table, heir_table, W1, b1, W2, b2, W3, b3):
    raise NotImplementedError("write your pallas kernel here")


# speedup vs baseline: 4.7119x; 4.7119x over previous
import jax
import jax.numpy as jnp
from jax import lax
from jax.experimental import pallas as pl
from jax.experimental.pallas import tpu as pltpu
from jax.experimental.pallas import tpu_sc as plsc

B = 4096
D = 32
VIS = 128
NC = 2
NS = 16
NW = NC * NS
BPW = B // NW
CH = 128

L_DE, L_SK, L_HE = 50, 8, 5
T_DE, T_SK, T_HE = BPW * L_DE, BPW * L_SK, BPW * L_HE
NCH_DE, NCH_SK, NCH_HE = T_DE // CH, T_SK // CH, T_HE // CH


def _sc_body(de_tab, sk_tab, he_tab, pr_tab,
             de_idx, sk_idx, he_idx, pr_idx,
             de_map, sk_map, he_map,
             de_out, sk_out, he_out, pr_out,
             idx_de_v, idx_sk_v, idx_he_v, idx_pr_v,
             map_de_v, map_sk_v, map_he_v,
             rows_v, acc_pr, zbuf,
             acc_de_sh, acc_sk_sh, acc_he_sh, gsem):
  cid = lax.axis_index("c")
  sid = lax.axis_index("s")
  wid = sid * NC + cid
  base = wid * BPW
  slab = sid * BPW

  pltpu.sync_copy(de_idx.at[pl.ds(base * L_DE, T_DE)], idx_de_v)
  pltpu.sync_copy(sk_idx.at[pl.ds(base * L_SK, T_SK)], idx_sk_v)
  pltpu.sync_copy(he_idx.at[pl.ds(base * L_HE, T_HE)], idx_he_v)
  pltpu.sync_copy(pr_idx.at[pl.ds(base, BPW)], idx_pr_v)
  pltpu.sync_copy(de_map.at[sid], map_de_v)
  pltpu.sync_copy(sk_map.at[sid], map_sk_v)
  pltpu.sync_copy(he_map.at[sid], map_he_v)

  z = jnp.zeros((16,), jnp.float32)

  @pl.loop(0, BPW)
  def _(r):
    zbuf[r, pl.ds(0, 16)] = z
    zbuf[r, pl.ds(16, 16)] = z

  pltpu.sync_copy(zbuf, acc_de_sh.at[pl.ds(slab, BPW)])
  pltpu.sync_copy(zbuf, acc_sk_sh.at[pl.ds(slab, BPW)])
  pltpu.sync_copy(zbuf, acc_he_sh.at[pl.ds(slab, BPW)])

  pltpu.async_copy(pr_tab.at[idx_pr_v], acc_pr, gsem).wait()

  def do_field(tab, idx_v, map_v, acc_sh, nch):
    @pl.loop(0, nch)
    def _(c):
      off = pl.multiple_of(c * CH, CH)
      pltpu.async_copy(tab.at[idx_v.at[pl.ds(off, CH)]], rows_v, gsem).wait()
      pltpu.sync_copy(rows_v, acc_sh.at[map_v.at[c]], add=True)

  do_field(de_tab, idx_de_v, map_de_v, acc_de_sh, NCH_DE)
  do_field(sk_tab, idx_sk_v, map_sk_v, acc_sk_sh, NCH_SK)
  do_field(he_tab, idx_he_v, map_he_v, acc_he_sh, NCH_HE)

  pltpu.sync_copy(acc_de_sh.at[pl.ds(slab, BPW)], de_out.at[pl.ds(base, BPW)])
  pltpu.sync_copy(acc_sk_sh.at[pl.ds(slab, BPW)], sk_out.at[pl.ds(base, BPW)])
  pltpu.sync_copy(acc_he_sh.at[pl.ds(slab, BPW)], he_out.at[pl.ds(base, BPW)])
  pltpu.sync_copy(acc_pr, pr_out.at[pl.ds(base, BPW)])


_sc_pool = pl.kernel(
    _sc_body,
    out_type=[jax.ShapeDtypeStruct((B, D), jnp.float32)] * 4,
    mesh=plsc.VectorSubcoreMesh(core_axis_name="c", subcore_axis_name="s",
                                num_cores=NC, num_subcores=NS),
    scratch_types=[
        pltpu.VMEM((T_DE,), jnp.int32),
        pltpu.VMEM((T_SK,), jnp.int32),
        pltpu.VMEM((T_HE,), jnp.int32),
        pltpu.VMEM((BPW,), jnp.int32),
        pltpu.VMEM((NCH_DE, CH), jnp.int32),
        pltpu.VMEM((NCH_SK, CH), jnp.int32),
        pltpu.VMEM((NCH_HE, CH), jnp.int32),
        pltpu.VMEM((CH, D), jnp.float32),
        pltpu.VMEM((BPW, D), jnp.float32),
        pltpu.VMEM((BPW, D), jnp.float32),
        pltpu.VMEM_SHARED((NS * BPW, D), jnp.float32),
        pltpu.VMEM_SHARED((NS * BPW, D), jnp.float32),
        pltpu.VMEM_SHARED((NS * BPW, D), jnp.float32),
        pltpu.SemaphoreType.DMA,
    ],
    compiler_params=pltpu.CompilerParams(use_tc_tiling_on_sc=False),
)


GRID = 8
TB = B // GRID


def _mlp_body(price, de_s, sk_s, he_s, pr_s,
              d_idx, s_idx, h_idx, p_idx, vis,
              de0, sk0, he0, pr0,
              w1p, w1e, w1v, b1, w2, b2, w3, b3, out):
  def pool(s_ref, idx_ref, r0_ref, ln):
    nz = jnp.sum((idx_ref[...] != 0).astype(jnp.float32), axis=1, keepdims=True)
    n0 = ln - nz
    return (s_ref[...] - n0 * r0_ref[...]) / jnp.maximum(nz, 1.0)

  pr = pool(pr_s, p_idx, pr0, 1.0)
  de = pool(de_s, d_idx, de0, float(L_DE))
  sk = pool(sk_s, s_idx, sk0, float(L_SK))
  he = pool(he_s, h_idx, he0, float(L_HE))
  emb = jnp.concatenate([pr, de, sk, he], axis=1)

  h = (price[...] * w1p[...]
       + jnp.dot(emb, w1e[...], preferred_element_type=jnp.float32)
       + jnp.dot(vis[...], w1v[...], preferred_element_type=jnp.float32)
       + b1[...])
  h = jnp.maximum(h, 0.0)
  h = jnp.maximum(jnp.dot(h, w2[...], preferred_element_type=jnp.float32) + b2[...], 0.0)
  out[...] = jnp.dot(h, w3[...], preferred_element_type=jnp.float32) + b3[...]


def _row_spec(cols):
  return pl.BlockSpec((TB, cols), lambda i: (i, 0))


def _fix_spec(r, c):
  return pl.BlockSpec((r, c), lambda i: (0, 0))


_mlp = pl.pallas_call(
    _mlp_body,
    grid=(GRID,),
    in_specs=[
        _row_spec(1),
        _row_spec(D), _row_spec(D), _row_spec(D), _row_spec(D),
        _row_spec(L_DE), _row_spec(L_SK), _row_spec(L_HE), _row_spec(1),
        _row_spec(VIS),
        _fix_spec(1, D), _fix_spec(1, D), _fix_spec(1, D), _fix_spec(1, D),
        _fix_spec(1, 256), _fix_spec(128, 256), _fix_spec(VIS, 256),
        _fix_spec(1, 256),
        _fix_spec(256, 128), _fix_spec(1, 128),
        _fix_spec(128, 64), _fix_spec(1, 64),
    ],
    out_specs=pl.BlockSpec((TB, 64), lambda i: (i, 0)),
    out_shape=jax.ShapeDtypeStruct((B, 64), jnp.float32),
)


def kernel(price_td, price_range_idx, description_idx, sku_idx, heir_idx,
           visual, price_range_table, desc_table, sku_table, heir_table,
           W1, b1, W2, b2, W3, b3):
  sids = (jnp.arange(NS, dtype=jnp.int32) * BPW)[:, None, None]
  de_map = (jnp.arange(T_DE, dtype=jnp.int32) // L_DE).reshape(1, NCH_DE, CH) + sids
  sk_map = (jnp.arange(T_SK, dtype=jnp.int32) // L_SK).reshape(1, NCH_SK, CH) + sids
  he_map = (jnp.arange(T_HE, dtype=jnp.int32) // L_HE).reshape(1, NCH_HE, CH) + sids

  de_s, sk_s, he_s, pr_s = _sc_pool(
      desc_table, sku_table, heir_table, price_range_table,
      description_idx.reshape(-1), sku_idx.reshape(-1),
      heir_idx.reshape(-1), price_range_idx.reshape(-1),
      de_map, sk_map, he_map)

  return _mlp(price_td, de_s, sk_s, he_s, pr_s,
              description_idx, sku_idx, heir_idx, price_range_idx, visual,
              desc_table[0:1], sku_table[0:1], heir_table[0:1],
              price_range_table[0:1],
              W1[0:1], W1[1:1 + 4 * D], W1[1 + 4 * D:], b1.reshape(1, -1),
              W2, b2.reshape(1, -1), W3, b3.reshape(1, -1))


# speedup vs baseline: 5.7139x; 1.2127x over previous
import jax
import jax.numpy as jnp
from jax import lax
from jax.experimental import pallas as pl
from jax.experimental.pallas import tpu as pltpu
from jax.experimental.pallas import tpu_sc as plsc

B = 4096
D = 32
VIS = 128
NC = 2
NS = 16
NW = NC * NS
BPW = B // NW
CH = 128

L_DE, L_SK, L_HE = 50, 8, 5
T_DE, T_SK, T_HE = BPW * L_DE, BPW * L_SK, BPW * L_HE
CH_DE, CH_SK, CH_HE = 1280, 512, 640
NCH_DE, NCH_SK, NCH_HE = T_DE // CH_DE, T_SK // CH_SK, T_HE // CH_HE
CH_MAX = max(CH_DE, CH_SK, CH_HE)


def _sc_body(de_tab, sk_tab, he_tab, pr_tab,
             de_idx, sk_idx, he_idx, pr_idx,
             de_map, sk_map, he_map,
             de_out, sk_out, he_out, pr_out,
             idx_de_v, idx_sk_v, idx_he_v, idx_pr_v,
             map_de_v, map_sk_v, map_he_v,
             buf0, buf1, acc_pr, zbuf,
             acc_de_sh, acc_sk_sh, acc_he_sh,
             gsem0, gsem1, ssem0, ssem1, psem, isem):
  cid = lax.axis_index("c")
  sid = lax.axis_index("s")
  wid = sid * NC + cid
  base = wid * BPW
  slab = sid * BPW

  st = [
      pltpu.async_copy(de_idx.at[pl.ds(base * L_DE, T_DE)], idx_de_v, isem),
      pltpu.async_copy(sk_idx.at[pl.ds(base * L_SK, T_SK)], idx_sk_v, isem),
      pltpu.async_copy(he_idx.at[pl.ds(base * L_HE, T_HE)], idx_he_v, isem),
      pltpu.async_copy(pr_idx.at[pl.ds(base, BPW)], idx_pr_v, isem),
      pltpu.async_copy(de_map.at[sid], map_de_v, isem),
      pltpu.async_copy(sk_map.at[sid], map_sk_v, isem),
      pltpu.async_copy(he_map.at[sid], map_he_v, isem),
  ]

  z = jnp.zeros((16,), jnp.float32)

  @pl.loop(0, BPW)
  def _(r):
    zbuf[r, pl.ds(0, 16)] = z
    zbuf[r, pl.ds(16, 16)] = z

  pltpu.sync_copy(zbuf, acc_de_sh.at[pl.ds(slab, BPW)])
  pltpu.sync_copy(zbuf, acc_sk_sh.at[pl.ds(slab, BPW)])
  pltpu.sync_copy(zbuf, acc_he_sh.at[pl.ds(slab, BPW)])
  for c in st:
    c.wait()

  pr_cp = pltpu.async_copy(pr_tab.at[idx_pr_v], acc_pr, psem)

  chunks = (
      [(de_tab, idx_de_v, c * CH_DE, CH_DE, map_de_v, c, acc_de_sh)
       for c in range(NCH_DE)]
      + [(sk_tab, idx_sk_v, c * CH_SK, CH_SK, map_sk_v, c, acc_sk_sh)
         for c in range(NCH_SK)]
      + [(he_tab, idx_he_v, c * CH_HE, CH_HE, map_he_v, c, acc_he_sh)
         for c in range(NCH_HE)]
  )
  bufs = (buf0, buf1)
  gsems = (gsem0, gsem1)
  ssems = (ssem0, ssem1)
  n = len(chunks)

  def fire_gather(c):
    tab, idx_v, off, ch, _, _, _ = chunks[c]
    b = c & 1
    return pltpu.async_copy(tab.at[idx_v.at[pl.ds(off, ch)]],
                            bufs[b].at[pl.ds(0, ch)], gsems[b])

  def fire_scatter(c):
    _, _, _, ch, map_v, mrow, acc_sh = chunks[c]
    b = c & 1
    return pltpu.async_copy(bufs[b].at[pl.ds(0, ch)],
                            acc_sh.at[map_v.at[mrow]], ssems[b], add=True)

  g = {0: fire_gather(0)}
  s = {}
  for c in range(n):
    g[c].wait()
    if c + 1 < n:
      if c - 1 >= 0:
        s[c - 1].wait()
      g[c + 1] = fire_gather(c + 1)
    s[c] = fire_scatter(c)
  if n >= 2:
    s[n - 2].wait()
  s[n - 1].wait()
  pr_cp.wait()

  pltpu.sync_copy(acc_de_sh.at[pl.ds(slab, BPW)], de_out.at[pl.ds(base, BPW)])
  pltpu.sync_copy(acc_sk_sh.at[pl.ds(slab, BPW)], sk_out.at[pl.ds(base, BPW)])
  pltpu.sync_copy(acc_he_sh.at[pl.ds(slab, BPW)], he_out.at[pl.ds(base, BPW)])
  pltpu.sync_copy(acc_pr, pr_out.at[pl.ds(base, BPW)])


_sc_pool = pl.kernel(
    _sc_body,
    out_type=[jax.ShapeDtypeStruct((B, D), jnp.float32)] * 4,
    mesh=plsc.VectorSubcoreMesh(core_axis_name="c", subcore_axis_name="s",
                                num_cores=NC, num_subcores=NS),
    scratch_types=[
        pltpu.VMEM((T_DE,), jnp.int32),
        pltpu.VMEM((T_SK,), jnp.int32),
        pltpu.VMEM((T_HE,), jnp.int32),
        pltpu.VMEM((BPW,), jnp.int32),
        pltpu.VMEM((NCH_DE, CH_DE), jnp.int32),
        pltpu.VMEM((NCH_SK, CH_SK), jnp.int32),
        pltpu.VMEM((NCH_HE, CH_HE), jnp.int32),
        pltpu.VMEM((CH_MAX, D), jnp.float32),
        pltpu.VMEM((CH_MAX, D), jnp.float32),
        pltpu.VMEM((BPW, D), jnp.float32),
        pltpu.VMEM((BPW, D), jnp.float32),
        pltpu.VMEM_SHARED((NS * BPW, D), jnp.float32),
        pltpu.VMEM_SHARED((NS * BPW, D), jnp.float32),
        pltpu.VMEM_SHARED((NS * BPW, D), jnp.float32),
        pltpu.SemaphoreType.DMA,
        pltpu.SemaphoreType.DMA,
        pltpu.SemaphoreType.DMA,
        pltpu.SemaphoreType.DMA,
        pltpu.SemaphoreType.DMA,
        pltpu.SemaphoreType.DMA,
    ],
    compiler_params=pltpu.CompilerParams(use_tc_tiling_on_sc=False),
)


GRID = 8
TB = B // GRID


def _mlp_body(price, de_s, sk_s, he_s, pr_s,
              d_idx, s_idx, h_idx, p_idx, vis,
              de0, sk0, he0, pr0,
              w1p, w1e, w1v, b1, w2, b2, w3, b3, out):
  def pool(s_ref, idx_ref, r0_ref, ln):
    nz = jnp.sum((idx_ref[...] != 0).astype(jnp.float32), axis=1, keepdims=True)
    n0 = ln - nz
    return (s_ref[...] - n0 * r0_ref[...]) / jnp.maximum(nz, 1.0)

  pr = pool(pr_s, p_idx, pr0, 1.0)
  de = pool(de_s, d_idx, de0, float(L_DE))
  sk = pool(sk_s, s_idx, sk0, float(L_SK))
  he = pool(he_s, h_idx, he0, float(L_HE))
  emb = jnp.concatenate([pr, de, sk, he], axis=1)

  h = (price[...] * w1p[...]
       + jnp.dot(emb, w1e[...], preferred_element_type=jnp.float32)
       + jnp.dot(vis[...], w1v[...], preferred_element_type=jnp.float32)
       + b1[...])
  h = jnp.maximum(h, 0.0)
  h = jnp.maximum(jnp.dot(h, w2[...], preferred_element_type=jnp.float32) + b2[...], 0.0)
  out[...] = jnp.dot(h, w3[...], preferred_element_type=jnp.float32) + b3[...]


def _row_spec(cols):
  return pl.BlockSpec((TB, cols), lambda i: (i, 0))


def _fix_spec(r, c):
  return pl.BlockSpec((r, c), lambda i: (0, 0))


_mlp = pl.pallas_call(
    _mlp_body,
    grid=(GRID,),
    in_specs=[
        _row_spec(1),
        _row_spec(D), _row_spec(D), _row_spec(D), _row_spec(D),
        _row_spec(L_DE), _row_spec(L_SK), _row_spec(L_HE), _row_spec(1),
        _row_spec(VIS),
        _fix_spec(1, D), _fix_spec(1, D), _fix_spec(1, D), _fix_spec(1, D),
        _fix_spec(1, 256), _fix_spec(128, 256), _fix_spec(VIS, 256),
        _fix_spec(1, 256),
        _fix_spec(256, 128), _fix_spec(1, 128),
        _fix_spec(128, 64), _fix_spec(1, 64),
    ],
    out_specs=pl.BlockSpec((TB, 64), lambda i: (i, 0)),
    out_shape=jax.ShapeDtypeStruct((B, 64), jnp.float32),
)


def kernel(price_td, price_range_idx, description_idx, sku_idx, heir_idx,
           visual, price_range_table, desc_table, sku_table, heir_table,
           W1, b1, W2, b2, W3, b3):
  sids = (jnp.arange(NS, dtype=jnp.int32) * BPW)[:, None, None]
  de_map = (jnp.arange(T_DE, dtype=jnp.int32) // L_DE).reshape(1, NCH_DE, CH_DE) + sids
  sk_map = (jnp.arange(T_SK, dtype=jnp.int32) // L_SK).reshape(1, NCH_SK, CH_SK) + sids
  he_map = (jnp.arange(T_HE, dtype=jnp.int32) // L_HE).reshape(1, NCH_HE, CH_HE) + sids

  de_s, sk_s, he_s, pr_s = _sc_pool(
      desc_table, sku_table, heir_table, price_range_table,
      description_idx.reshape(-1), sku_idx.reshape(-1),
      heir_idx.reshape(-1), price_range_idx.reshape(-1),
      de_map, sk_map, he_map)

  return _mlp(price_td, de_s, sk_s, he_s, pr_s,
              description_idx, sku_idx, heir_idx, price_range_idx, visual,
              desc_table[0:1], sku_table[0:1], heir_table[0:1],
              price_range_table[0:1],
              W1[0:1], W1[1:1 + 4 * D], W1[1 + 4 * D:], b1.reshape(1, -1),
              W2, b2.reshape(1, -1), W3, b3.reshape(1, -1))


# speedup vs baseline: 6.1804x; 1.0816x over previous
import jax
import jax.numpy as jnp
from jax import lax
from jax.experimental import pallas as pl
from jax.experimental.pallas import tpu as pltpu
from jax.experimental.pallas import tpu_sc as plsc

B = 4096
D = 32
VIS = 128
NC = 2
NS = 16
NW = NC * NS
BPW = B // NW

L_DE, L_SK, L_HE = 50, 8, 5
T_DE, T_SK, T_HE = BPW * L_DE, BPW * L_SK, BPW * L_HE

IT_DE, IT_SK, IT_HE, IT_PR = 16, 64, 128, 128
NCH_DE, NCH_SK = BPW // IT_DE, BPW // IT_SK
ROWS_MAX = max(IT_DE * L_DE, IT_SK * L_SK, IT_HE * L_HE, IT_PR)
NBUF = 3


def _sum_chunk(buf, sums, item_base, n_items, ln):
  @pl.loop(0, n_items)
  def _(i):
    r0 = i * ln
    a0 = buf[r0, pl.ds(0, 16)]
    a1 = buf[r0, pl.ds(16, 16)]
    for t in range(1, ln):
      a0 = a0 + buf[r0 + t, pl.ds(0, 16)]
      a1 = a1 + buf[r0 + t, pl.ds(16, 16)]
    row = item_base + i
    sums[row, pl.ds(0, 16)] = a0
    sums[row, pl.ds(16, 16)] = a1


def _run_chunks(chunks, bufs, gsems):
  n = len(chunks)

  def fire(c):
    tab, idx_v, off, rows, _, _, _, _ = chunks[c]
    b = c % NBUF
    return pltpu.async_copy(tab.at[idx_v.at[pl.ds(off, rows)]],
                            bufs[b].at[pl.ds(0, rows)], gsems[b])

  g = {}
  for c in range(min(NBUF, n)):
    g[c] = fire(c)
  for c in range(n):
    g[c].wait()
    _, _, _, _, sums, item_base, n_items, ln = chunks[c]
    _sum_chunk(bufs[c % NBUF], sums, item_base, n_items, ln)
    if c + NBUF < n:
      g[c + NBUF] = fire(c + NBUF)


def _wid_base():
  cid = lax.axis_index("c")
  sid = lax.axis_index("s")
  return (sid * NC + cid) * BPW


def _sc_desc_body(de_tab, de_idx, de_out, idx_de_v, b0, b1, b2,
                  sums_de, s0, s1, s2, osem):
  base = _wid_base()
  pltpu.sync_copy(de_idx.at[pl.ds(base * L_DE, T_DE)], idx_de_v)
  chunks = [
      (de_tab, idx_de_v, c * IT_DE * L_DE, IT_DE * L_DE,
       sums_de, c * IT_DE, IT_DE, L_DE)
      for c in range(NCH_DE)
  ]
  _run_chunks(chunks, (b0, b1, b2), (s0, s1, s2))
  pltpu.async_copy(sums_de, de_out.at[pl.ds(base, BPW)], osem).wait()


def _sc_rest_body(sk_tab, he_tab, pr_tab, sk_idx, he_idx, pr_idx,
                  sk_out, he_out, pr_out,
                  idx_sk_v, idx_he_v, idx_pr_v, b0, b1, b2,
                  sums_sk, sums_he, sums_pr, s0, s1, s2, osem):
  base = _wid_base()
  st = [
      pltpu.async_copy(sk_idx.at[pl.ds(base * L_SK, T_SK)], idx_sk_v, osem),
      pltpu.async_copy(he_idx.at[pl.ds(base * L_HE, T_HE)], idx_he_v, osem),
      pltpu.async_copy(pr_idx.at[pl.ds(base, BPW)], idx_pr_v, osem),
  ]
  for c in st:
    c.wait()
  chunks = (
      [(sk_tab, idx_sk_v, c * IT_SK * L_SK, IT_SK * L_SK,
        sums_sk, c * IT_SK, IT_SK, L_SK) for c in range(NCH_SK)]
      + [(he_tab, idx_he_v, 0, T_HE, sums_he, 0, IT_HE, L_HE)]
      + [(pr_tab, idx_pr_v, 0, BPW, sums_pr, 0, IT_PR, 1)]
  )
  _run_chunks(chunks, (b0, b1, b2), (s0, s1, s2))
  ocp = [
      pltpu.async_copy(sums_sk, sk_out.at[pl.ds(base, BPW)], osem),
      pltpu.async_copy(sums_he, he_out.at[pl.ds(base, BPW)], osem),
      pltpu.async_copy(sums_pr, pr_out.at[pl.ds(base, BPW)], osem),
  ]
  for c in ocp:
    c.wait()


_MESH = plsc.VectorSubcoreMesh(core_axis_name="c", subcore_axis_name="s",
                               num_cores=NC, num_subcores=NS)
_SC_PARAMS = pltpu.CompilerParams(use_tc_tiling_on_sc=False)

_sc_desc = pl.kernel(
    _sc_desc_body,
    out_type=jax.ShapeDtypeStruct((B, D), jnp.float32),
    mesh=_MESH,
    scratch_types=[
        pltpu.VMEM((T_DE,), jnp.int32),
        pltpu.VMEM((ROWS_MAX, D), jnp.float32),
        pltpu.VMEM((ROWS_MAX, D), jnp.float32),
        pltpu.VMEM((ROWS_MAX, D), jnp.float32),
        pltpu.VMEM((BPW, D), jnp.float32),
        pltpu.SemaphoreType.DMA,
        pltpu.SemaphoreType.DMA,
        pltpu.SemaphoreType.DMA,
        pltpu.SemaphoreType.DMA,
    ],
    compiler_params=_SC_PARAMS,
)

_sc_rest = pl.kernel(
    _sc_rest_body,
    out_type=[jax.ShapeDtypeStruct((B, D), jnp.float32)] * 3,
    mesh=_MESH,
    scratch_types=[
        pltpu.VMEM((T_SK,), jnp.int32),
        pltpu.VMEM((T_HE,), jnp.int32),
        pltpu.VMEM((BPW,), jnp.int32),
        pltpu.VMEM((ROWS_MAX, D), jnp.float32),
        pltpu.VMEM((ROWS_MAX, D), jnp.float32),
        pltpu.VMEM((ROWS_MAX, D), jnp.float32),
        pltpu.VMEM((BPW, D), jnp.float32),
        pltpu.VMEM((BPW, D), jnp.float32),
        pltpu.VMEM((BPW, D), jnp.float32),
        pltpu.SemaphoreType.DMA,
        pltpu.SemaphoreType.DMA,
        pltpu.SemaphoreType.DMA,
        pltpu.SemaphoreType.DMA,
    ],
    compiler_params=_SC_PARAMS,
)


GRID = 8
TB = B // GRID


def _mlp_body(price, de_s, sk_s, he_s, pr_s,
              d_idx, s_idx, h_idx, p_idx, vis,
              de0, sk0, he0, pr0,
              w1p, w1e, w1v, b1, w2, b2, w3, b3, out):
  def pool(s_ref, idx_ref, r0_ref, ln):
    nz = jnp.sum((idx_ref[...] != 0).astype(jnp.float32), axis=1, keepdims=True)
    n0 = ln - nz
    return (s_ref[...] - n0 * r0_ref[...]) / jnp.maximum(nz, 1.0)

  pr = pool(pr_s, p_idx, pr0, 1.0)
  de = pool(de_s, d_idx, de0, float(L_DE))
  sk = pool(sk_s, s_idx, sk0, float(L_SK))
  he = pool(he_s, h_idx, he0, float(L_HE))
  emb = jnp.concatenate([pr, de, sk, he], axis=1)

  h = (price[...] * w1p[...]
       + jnp.dot(emb, w1e[...], preferred_element_type=jnp.float32)
       + jnp.dot(vis[...], w1v[...], preferred_element_type=jnp.float32)
       + b1[...])
  h = jnp.maximum(h, 0.0)
  h = jnp.maximum(jnp.dot(h, w2[...], preferred_element_type=jnp.float32) + b2[...], 0.0)
  out[...] = jnp.dot(h, w3[...], preferred_element_type=jnp.float32) + b3[...]


def _row_spec(cols):
  return pl.BlockSpec((TB, cols), lambda i: (i, 0))


def _fix_spec(r, c):
  return pl.BlockSpec((r, c), lambda i: (0, 0))


_mlp = pl.pallas_call(
    _mlp_body,
    grid=(GRID,),
    in_specs=[
        _row_spec(1),
        _row_spec(D), _row_spec(D), _row_spec(D), _row_spec(D),
        _row_spec(L_DE), _row_spec(L_SK), _row_spec(L_HE), _row_spec(1),
        _row_spec(VIS),
        _fix_spec(1, D), _fix_spec(1, D), _fix_spec(1, D), _fix_spec(1, D),
        _fix_spec(1, 256), _fix_spec(128, 256), _fix_spec(VIS, 256),
        _fix_spec(1, 256),
        _fix_spec(256, 128), _fix_spec(1, 128),
        _fix_spec(128, 64), _fix_spec(1, 64),
    ],
    out_specs=pl.BlockSpec((TB, 64), lambda i: (i, 0)),
    out_shape=jax.ShapeDtypeStruct((B, 64), jnp.float32),
)


def kernel(price_td, price_range_idx, description_idx, sku_idx, heir_idx,
           visual, price_range_table, desc_table, sku_table, heir_table,
           W1, b1, W2, b2, W3, b3):
  de_s = _sc_desc(desc_table, description_idx.reshape(-1))
  sk_s, he_s, pr_s = _sc_rest(
      sku_table, heir_table, price_range_table,
      sku_idx.reshape(-1), heir_idx.reshape(-1), price_range_idx.reshape(-1))

  return _mlp(price_td, de_s, sk_s, he_s, pr_s,
              description_idx, sku_idx, heir_idx, price_range_idx, visual,
              desc_table[0:1], sku_table[0:1], heir_table[0:1],
              price_range_table[0:1],
              W1[0:1], W1[1:1 + 4 * D], W1[1 + 4 * D:], b1.reshape(1, -1),
              W2, b2.reshape(1, -1), W3, b3.reshape(1, -1))
